# baseline (speedup 1.0000x reference)
import jax
import jax.numpy as jnp
import numpy as np
from jax import lax
from jax.experimental import pallas as pl
from jax.experimental.pallas import tpu as pltpu
from jax.experimental.pallas import tpu_sc as plsc

_VOCAB = 1000000
_D = 128
_SEQ = 8192

_NC = 2
_NS = 16
_L = 16
_NW = _NC * _NS
_B_PER_W = _SEQ // _NW
_K = 2
_R = _B_PER_W // _K


def _pos_table(d_model, max_len):
    pos = np.arange(max_len, dtype=np.float32)[:, None]
    _2i = np.arange(0, d_model, 2, dtype=np.float32)
    angle = pos / np.float32(10000.0) ** (_2i / np.float32(d_model))
    table = np.zeros((max_len, d_model), dtype=np.float32)
    table[:, 0::2] = np.sin(angle)
    table[:, 1::2] = np.cos(angle)
    return table


_POS_FLAT = jnp.asarray(_pos_table(_D, _SEQ).reshape(-1))


def _body(tok_hbm, idx_hbm, pos_hbm, out_hbm,
          idx_v, rows_v, pos_v, gsem0, gsem1, psem, wsem):
    wid = lax.axis_index("s") * _NC + lax.axis_index("c")
    base = wid * _B_PER_W

    pltpu.sync_copy(idx_hbm.at[pl.ds(base, _B_PER_W)], idx_v)
    g0 = pltpu.async_copy(tok_hbm.at[idx_v.at[pl.ds(0, _R)]],
                          rows_v.at[0], gsem0)
    g1 = pltpu.async_copy(tok_hbm.at[idx_v.at[pl.ds(_R, _R)]],
                          rows_v.at[1], gsem1)
    pg = pltpu.async_copy(pos_hbm.at[pl.ds(base * _D, _B_PER_W * _D)],
                          pos_v, psem)
    pg.wait()
    g0.wait()

    def add_rows(k):
        def add_row(r, carry):
            off = (k * _R + r) * _D
            for c in range(_D // _L):
                plsc.addupdate(rows_v.at[k, r, pl.ds(c * _L, _L)],
                               pos_v[pl.ds(off + c * _L, _L)])
            return carry
        lax.fori_loop(0, _R, add_row, 0, unroll=2)

    add_rows(0)
    w0 = pltpu.async_copy(rows_v.at[0],
                          out_hbm.at[pl.ds(base, _R)], wsem)
    g1.wait()
    add_rows(1)
    w1 = pltpu.async_copy(rows_v.at[1],
                          out_hbm.at[pl.ds(base + _R, _R)], wsem)
    w0.wait()
    w1.wait()


def _embed(x, tok_table, pos):
    mesh = plsc.VectorSubcoreMesh(
        core_axis_name="c", subcore_axis_name="s",
        num_cores=_NC, num_subcores=_NS)
    return pl.kernel(
        _body,
        out_type=jax.ShapeDtypeStruct((_SEQ, _D), jnp.float32),
        mesh=mesh,
        scratch_types=[
            pltpu.VMEM((_B_PER_W,), jnp.int32),
            pltpu.VMEM((_K, _R, _D), jnp.float32),
            pltpu.VMEM((_B_PER_W * _D,), jnp.float32),
            pltpu.SemaphoreType.DMA,
            pltpu.SemaphoreType.DMA,
            pltpu.SemaphoreType.DMA,
            pltpu.SemaphoreType.DMA,
        ],
    )(tok_table, x, pos)


def kernel(x, tok_table):
    return _embed(x.astype(jnp.int32), tok_table, _POS_FLAT)
